# Initial kernel scaffold; baseline (speedup 1.0000x reference)
import jax
import jax.numpy as jnp
from jax.experimental import pallas as pl


def kernel(x, edge_index, edge_type, relation_embs, W1, Ws1, Wr1, W2, Ws2, Wr2):
    raise NotImplementedError("write your pallas kernel here")


# speedup vs baseline: 1.6041x; 1.6041x over previous
import functools

import jax
import jax.numpy as jnp
from jax import lax
from jax.experimental import pallas as pl
from jax.experimental.pallas import tpu as pltpu
from jax.experimental.pallas import tpu_sc as plsc

N = 10000
E = 320000
D = 128
R = 100

NC = 2
NS = 16
NW = NC * NS
C = 128
EPW = 10240
NCH = EPW // C
EPAD = NW * EPW
NPAD = 10240
RPS = NPAD // NS
ZCH = RPS // C


def _sc_layer_body(first, x_h, rel_h, src_h, dst_h, typ_h, inv_h, agg_out,
                   inv_out, agg_sh, deg_sh, src_v, dst_v, typ_v, xr_v, rr_v,
                   ones_v, inv_v):
    cid = lax.axis_index("c")
    sid = lax.axis_index("s")
    w = sid * NC + cid

    zero16 = jnp.zeros((16,), jnp.float32)
    one16 = jnp.ones((16,), jnp.float32)

    @pl.loop(0, C, step=16)
    def _(r):
        ones_v[pl.ds(r, 16)] = one16

    @pl.loop(0, C)
    def _(r):
        @pl.loop(0, D, step=16)
        def _(j):
            xr_v[r, pl.ds(j, 16)] = zero16

    @pl.loop(0, ZCH)
    def _(k):
        r0 = sid * RPS + k * C
        pltpu.sync_copy(xr_v, agg_sh.at[pl.ds(r0, C)])
        if first:
            pltpu.sync_copy(xr_v.at[0], deg_sh.at[pl.ds(r0, C)])

    plsc.subcore_barrier()

    @pl.loop(0, NCH)
    def _(i):
        base = w * EPW + i * C
        pltpu.sync_copy(src_h.at[pl.ds(base, C)], src_v)
        pltpu.sync_copy(typ_h.at[pl.ds(base, C)], typ_v)
        pltpu.sync_copy(dst_h.at[pl.ds(base, C)], dst_v)
        pltpu.sync_copy(x_h.at[src_v], xr_v)
        pltpu.sync_copy(rel_h.at[typ_v], rr_v)

        @pl.loop(0, C)
        def _(r):
            @pl.loop(0, D, step=16)
            def _(j):
                xr_v[r, pl.ds(j, 16)] = (
                    xr_v[r, pl.ds(j, 16)] * rr_v[r, pl.ds(j, 16)]
                )

        pltpu.sync_copy(xr_v, agg_sh.at[dst_v], add=True)
        if first:
            pltpu.sync_copy(ones_v, deg_sh.at[dst_v], add=True)

    if first:
        w2 = sid * NC + (1 - cid)

        @pl.loop(0, NCH)
        def _(i):
            base = w2 * EPW + i * C
            pltpu.sync_copy(dst_h.at[pl.ds(base, C)], dst_v)
            pltpu.sync_copy(ones_v, deg_sh.at[dst_v], add=True)

    plsc.subcore_barrier()

    @pl.loop(0, ZCH)
    def _(k):
        r0 = sid * RPS + k * C
        pltpu.sync_copy(agg_sh.at[pl.ds(r0, C)], xr_v)
        if first:
            pltpu.sync_copy(deg_sh.at[pl.ds(r0, C)], inv_v)

            @pl.loop(0, C, step=16)
            def _(j):
                inv_v[pl.ds(j, 16)] = 1.0 / jnp.maximum(
                    inv_v[pl.ds(j, 16)], 1.0)
        else:
            pltpu.sync_copy(inv_h.at[pl.ds(r0, C)], inv_v)

        @pl.loop(0, C, step=16)
        def _(g):
            iv16 = inv_v[pl.ds(g, 16)]

            @pl.loop(0, 16)
            def _(t):
                ivv = lax.gather(
                    iv16, jnp.full((16, 1), t, jnp.int32),
                    lax.GatherDimensionNumbers(
                        offset_dims=(), collapsed_slice_dims=(0,),
                        start_index_map=(0,)),
                    slice_sizes=(1,),
                    mode=lax.GatherScatterMode.PROMISE_IN_BOUNDS)

                @pl.loop(0, D, step=16)
                def _(j):
                    xr_v[g + t, pl.ds(j, 16)] = (
                        xr_v[g + t, pl.ds(j, 16)] * ivv)

        pltpu.sync_copy(xr_v, agg_out.at[cid, pl.ds(r0, C)])
        if first:
            @pl.when(cid == 0)
            def _():
                pltpu.sync_copy(inv_v, inv_out.at[pl.ds(r0, C)])


def _make_sc_layer(first):
    mesh = plsc.VectorSubcoreMesh(core_axis_name="c", subcore_axis_name="s")
    return pl.kernel(
        functools.partial(_sc_layer_body, first),
        out_type=[jax.ShapeDtypeStruct((NC, NPAD, D), jnp.float32),
                  jax.ShapeDtypeStruct((NPAD,), jnp.float32)],
        mesh=mesh,
        scratch_types=[
            pltpu.VMEM_SHARED((NPAD, D), jnp.float32),
            pltpu.VMEM_SHARED((NPAD,), jnp.float32),
            pltpu.VMEM((C,), jnp.int32),
            pltpu.VMEM((C,), jnp.int32),
            pltpu.VMEM((C,), jnp.int32),
            pltpu.VMEM((C, D), jnp.float32),
            pltpu.VMEM((C, D), jnp.float32),
            pltpu.VMEM((C,), jnp.float32),
            pltpu.VMEM((C,), jnp.float32),
        ],
    )


_sc_layer_first = _make_sc_layer(True)
_sc_layer_second = _make_sc_layer(False)


def _rel_body(rel_ref, wr1_ref, wr2_ref, rel2_ref, relout_ref):
    r2 = jnp.dot(rel_ref[...], wr1_ref[...], preferred_element_type=jnp.float32)
    rel2_ref[...] = r2
    relout_ref[...] = jnp.dot(r2, wr2_ref[...],
                              preferred_element_type=jnp.float32)


_rel_update = pl.pallas_call(
    _rel_body,
    out_shape=[jax.ShapeDtypeStruct((R, D), jnp.float32),
               jax.ShapeDtypeStruct((R, D), jnp.float32)],
)

BN = 1024
NB = NPAD // BN


def _dense_body(p_ref, xin_ref, w_ref, ws_ref, h_ref):
    agg = p_ref[0] + p_ref[1]
    xin = xin_ref[...]
    h = (jnp.dot(agg, w_ref[...], preferred_element_type=jnp.float32)
         + jnp.dot(xin, ws_ref[...], preferred_element_type=jnp.float32))
    h_ref[...] = jnp.maximum(h, 0.0) + xin


_dense_layer = pl.pallas_call(
    _dense_body,
    grid=(NB,),
    in_specs=[
        pl.BlockSpec((NC, BN, D), lambda i: (0, i, 0)),
        pl.BlockSpec((BN, D), lambda i: (i, 0)),
        pl.BlockSpec((D, D), lambda i: (0, 0)),
        pl.BlockSpec((D, D), lambda i: (0, 0)),
    ],
    out_specs=pl.BlockSpec((BN, D), lambda i: (i, 0)),
    out_shape=jax.ShapeDtypeStruct((NPAD, D), jnp.float32),
)


def kernel(x, edge_index, edge_type, relation_embs, W1, Ws1, Wr1, W2, Ws2, Wr2):
    pad = EPAD - E
    src = jnp.concatenate([edge_index[0], jnp.zeros((pad,), jnp.int32)])
    dst = jnp.concatenate([edge_index[1], jnp.full((pad,), N, jnp.int32)])
    typ = jnp.concatenate([edge_type, jnp.zeros((pad,), jnp.int32)])
    x_p = jnp.concatenate([x, jnp.zeros((NPAD - N, D), jnp.float32)])

    rel2, rel_out = _rel_update(relation_embs, Wr1, Wr2)

    dummy_inv = jnp.zeros((NPAD,), jnp.float32)
    p1, inv = _sc_layer_first(x_p, relation_embs, src, dst, typ, dummy_inv)
    h1 = _dense_layer(p1, x_p, W1, Ws1)

    p2, _ = _sc_layer_second(h1, rel2, src, dst, typ, inv)
    h2 = _dense_layer(p2, h1, W2, Ws2)

    return (h2[:N], rel_out)


# speedup vs baseline: 1.6042x; 1.0001x over previous
import functools

import jax
import jax.numpy as jnp
from jax import lax
from jax.experimental import pallas as pl
from jax.experimental.pallas import tpu as pltpu
from jax.experimental.pallas import tpu_sc as plsc

N = 10000
E = 320000
D = 128
R = 100

NC = 2
NS = 16
NW = NC * NS
C = 128
EPW = 10240
NCH = EPW // C
EPAD = NW * EPW
NPAD = 10240
RPS = NPAD // NS
ZCH = RPS // C


def _sc_layer_body(first, x_h, rel_h, src_h, dst_h, typ_h, inv_h, agg_out,
                   inv_out, agg_sh, deg_sh, src_v, dst_v, typ_v, xr_v, rr_v,
                   ones_v, inv_v):
    cid = lax.axis_index("c")
    sid = lax.axis_index("s")
    w = sid * NC + cid

    zero16 = jnp.zeros((16,), jnp.float32)
    one16 = jnp.ones((16,), jnp.float32)

    @pl.loop(0, C, step=16)
    def _(r):
        ones_v[pl.ds(r, 16)] = one16

    @pl.loop(0, C)
    def _(r):
        for j in range(0, D, 16):
            xr_v[r, pl.ds(j, 16)] = zero16

    @pl.loop(0, ZCH)
    def _(k):
        r0 = sid * RPS + k * C
        pltpu.sync_copy(xr_v, agg_sh.at[pl.ds(r0, C)])
        if first:
            pltpu.sync_copy(xr_v.at[0], deg_sh.at[pl.ds(r0, C)])

    plsc.subcore_barrier()

    @pl.loop(0, NCH)
    def _(i):
        base = w * EPW + i * C
        pltpu.sync_copy(src_h.at[pl.ds(base, C)], src_v)
        pltpu.sync_copy(typ_h.at[pl.ds(base, C)], typ_v)
        pltpu.sync_copy(dst_h.at[pl.ds(base, C)], dst_v)
        pltpu.sync_copy(x_h.at[src_v], xr_v)
        pltpu.sync_copy(rel_h.at[typ_v], rr_v)

        @pl.loop(0, C)
        def _(r):
            for j in range(0, D, 16):
                xr_v[r, pl.ds(j, 16)] = (
                    xr_v[r, pl.ds(j, 16)] * rr_v[r, pl.ds(j, 16)]
                )

        pltpu.sync_copy(xr_v, agg_sh.at[dst_v], add=True)
        if first:
            pltpu.sync_copy(ones_v, deg_sh.at[dst_v], add=True)

    if first:
        w2 = sid * NC + (1 - cid)

        @pl.loop(0, NCH)
        def _(i):
            base = w2 * EPW + i * C
            pltpu.sync_copy(dst_h.at[pl.ds(base, C)], dst_v)
            pltpu.sync_copy(ones_v, deg_sh.at[dst_v], add=True)

    plsc.subcore_barrier()

    @pl.loop(0, ZCH)
    def _(k):
        r0 = sid * RPS + k * C
        pltpu.sync_copy(agg_sh.at[pl.ds(r0, C)], xr_v)
        if first:
            pltpu.sync_copy(deg_sh.at[pl.ds(r0, C)], inv_v)

            @pl.loop(0, C, step=16)
            def _(j):
                inv_v[pl.ds(j, 16)] = 1.0 / jnp.maximum(
                    inv_v[pl.ds(j, 16)], 1.0)
        else:
            pltpu.sync_copy(inv_h.at[pl.ds(r0, C)], inv_v)

        @pl.loop(0, C, step=16)
        def _(g):
            iv16 = inv_v[pl.ds(g, 16)]

            @pl.loop(0, 16)
            def _(t):
                ivv = lax.gather(
                    iv16, jnp.full((16, 1), t, jnp.int32),
                    lax.GatherDimensionNumbers(
                        offset_dims=(), collapsed_slice_dims=(0,),
                        start_index_map=(0,)),
                    slice_sizes=(1,),
                    mode=lax.GatherScatterMode.PROMISE_IN_BOUNDS)

                for j in range(0, D, 16):
                    xr_v[g + t, pl.ds(j, 16)] = (
                        xr_v[g + t, pl.ds(j, 16)] * ivv)

        pltpu.sync_copy(xr_v, agg_out.at[cid, pl.ds(r0, C)])
        if first:
            @pl.when(cid == 0)
            def _():
                pltpu.sync_copy(inv_v, inv_out.at[pl.ds(r0, C)])


def _make_sc_layer(first):
    mesh = plsc.VectorSubcoreMesh(core_axis_name="c", subcore_axis_name="s")
    return pl.kernel(
        functools.partial(_sc_layer_body, first),
        out_type=[jax.ShapeDtypeStruct((NC, NPAD, D), jnp.float32),
                  jax.ShapeDtypeStruct((NPAD,), jnp.float32)],
        mesh=mesh,
        scratch_types=[
            pltpu.VMEM_SHARED((NPAD, D), jnp.float32),
            pltpu.VMEM_SHARED((NPAD,), jnp.float32),
            pltpu.VMEM((C,), jnp.int32),
            pltpu.VMEM((C,), jnp.int32),
            pltpu.VMEM((C,), jnp.int32),
            pltpu.VMEM((C, D), jnp.float32),
            pltpu.VMEM((C, D), jnp.float32),
            pltpu.VMEM((C,), jnp.float32),
            pltpu.VMEM((C,), jnp.float32),
        ],
    )


_sc_layer_first = _make_sc_layer(True)
_sc_layer_second = _make_sc_layer(False)


def _rel_body(rel_ref, wr1_ref, wr2_ref, rel2_ref, relout_ref):
    r2 = jnp.dot(rel_ref[...], wr1_ref[...], preferred_element_type=jnp.float32)
    rel2_ref[...] = r2
    relout_ref[...] = jnp.dot(r2, wr2_ref[...],
                              preferred_element_type=jnp.float32)


_rel_update = pl.pallas_call(
    _rel_body,
    out_shape=[jax.ShapeDtypeStruct((R, D), jnp.float32),
               jax.ShapeDtypeStruct((R, D), jnp.float32)],
)

BN = 1024
NB = NPAD // BN


def _dense_body(p_ref, xin_ref, w_ref, ws_ref, h_ref):
    agg = p_ref[0] + p_ref[1]
    xin = xin_ref[...]
    h = (jnp.dot(agg, w_ref[...], preferred_element_type=jnp.float32)
         + jnp.dot(xin, ws_ref[...], preferred_element_type=jnp.float32))
    h_ref[...] = jnp.maximum(h, 0.0) + xin


_dense_layer = pl.pallas_call(
    _dense_body,
    grid=(NB,),
    in_specs=[
        pl.BlockSpec((NC, BN, D), lambda i: (0, i, 0)),
        pl.BlockSpec((BN, D), lambda i: (i, 0)),
        pl.BlockSpec((D, D), lambda i: (0, 0)),
        pl.BlockSpec((D, D), lambda i: (0, 0)),
    ],
    out_specs=pl.BlockSpec((BN, D), lambda i: (i, 0)),
    out_shape=jax.ShapeDtypeStruct((NPAD, D), jnp.float32),
)


def kernel(x, edge_index, edge_type, relation_embs, W1, Ws1, Wr1, W2, Ws2, Wr2):
    pad = EPAD - E
    src = jnp.concatenate([edge_index[0], jnp.zeros((pad,), jnp.int32)])
    dst = jnp.concatenate([edge_index[1], jnp.full((pad,), N, jnp.int32)])
    typ = jnp.concatenate([edge_type, jnp.zeros((pad,), jnp.int32)])
    x_p = jnp.concatenate([x, jnp.zeros((NPAD - N, D), jnp.float32)])

    rel2, rel_out = _rel_update(relation_embs, Wr1, Wr2)

    dummy_inv = jnp.zeros((NPAD,), jnp.float32)
    p1, inv = _sc_layer_first(x_p, relation_embs, src, dst, typ, dummy_inv)
    h1 = _dense_layer(p1, x_p, W1, Ws1)

    p2, _ = _sc_layer_second(h1, rel2, src, dst, typ, inv)
    h2 = _dense_layer(p2, h1, W2, Ws2)

    return (h2[:N], rel_out)


# speedup vs baseline: 2.9221x; 1.8215x over previous
import functools

import jax
import jax.numpy as jnp
from jax import lax
from jax.experimental import pallas as pl
from jax.experimental.pallas import tpu as pltpu
from jax.experimental.pallas import tpu_sc as plsc

N = 10000
E = 320000
D = 128
R = 100

NC = 2
NS = 16
NW = NC * NS
C = 80
EPW = 10240
NCH = EPW // C
EPAD = NW * EPW
NPAD = 10240
RPS = NPAD // NS
ZCH = RPS // C


def _sc_layer_body(first, x_h, rel_h, src_h, dst_h, typ_h, inv_h, agg_out,
                   inv_out, agg_sh, deg_sh, src_v, dst_v, typ_v, xr_v, rr_v,
                   ones_v, inv_v, isem0, isem1, gx0, gx1, gr0, gr1, ss0, ss1,
                   ds0, ds1):
    cid = lax.axis_index("c")
    sid = lax.axis_index("s")
    w = sid * NC + cid

    isem = (isem0, isem1)
    gxs = (gx0, gx1)
    grs = (gr0, gr1)
    sss = (ss0, ss1)
    dss = (ds0, ds1)

    zero16 = jnp.zeros((16,), jnp.float32)
    one16 = jnp.ones((16,), jnp.float32)

    def issue_idx(i, b):
        base = w * EPW + i * C
        pltpu.async_copy(src_h.at[pl.ds(base, C)], src_v.at[b], isem[b])
        pltpu.async_copy(typ_h.at[pl.ds(base, C)], typ_v.at[b], isem[b])
        pltpu.async_copy(dst_h.at[pl.ds(base, C)], dst_v.at[b], isem[b])

    def wait_idx(b):
        pltpu.make_async_copy(src_h.at[pl.ds(0, C)], src_v.at[b],
                              isem[b]).wait()
        pltpu.make_async_copy(typ_h.at[pl.ds(0, C)], typ_v.at[b],
                              isem[b]).wait()
        pltpu.make_async_copy(dst_h.at[pl.ds(0, C)], dst_v.at[b],
                              isem[b]).wait()

    def issue_gathers(b):
        pltpu.async_copy(x_h.at[src_v.at[b]], xr_v.at[b], gxs[b])
        pltpu.async_copy(rel_h.at[typ_v.at[b]], rr_v.at[b], grs[b])

    def wait_gathers(b):
        pltpu.make_async_copy(x_h.at[pl.ds(0, C)], xr_v.at[b], gxs[b]).wait()
        pltpu.make_async_copy(x_h.at[pl.ds(0, C)], rr_v.at[b], grs[b]).wait()

    def multiply(b):
        @pl.loop(0, C)
        def _(r):
            for j in range(0, D, 16):
                xr_v[b, r, pl.ds(j, 16)] = (
                    xr_v[b, r, pl.ds(j, 16)] * rr_v[b, r, pl.ds(j, 16)]
                )

    def issue_scatter(b):
        pltpu.async_copy(xr_v.at[b], agg_sh.at[dst_v.at[b]], sss[b], add=True)
        if first:
            pltpu.async_copy(ones_v, deg_sh.at[dst_v.at[b]], dss[b], add=True)

    def wait_scatter(b):
        pltpu.make_async_copy(x_h.at[pl.ds(0, C)], xr_v.at[b], sss[b]).wait()
        if first:
            pltpu.make_async_copy(inv_h.at[pl.ds(0, C)], ones_v,
                                  dss[b]).wait()

    issue_idx(0, 0)

    @pl.loop(0, C, step=16)
    def _(r):
        ones_v[pl.ds(r, 16)] = one16
        inv_v[pl.ds(r, 16)] = zero16

    @pl.loop(0, C)
    def _(r):
        for j in range(0, D, 16):
            xr_v[1, r, pl.ds(j, 16)] = zero16

    @pl.loop(0, ZCH)
    def _(k):
        r0 = sid * RPS + k * C
        pltpu.sync_copy(xr_v.at[1], agg_sh.at[pl.ds(r0, C)])
        if first:
            pltpu.sync_copy(inv_v, deg_sh.at[pl.ds(r0, C)])

    plsc.subcore_barrier()

    @pl.loop(0, NCH, step=2)
    def _(ii):
        for b in range(2):
            i = ii + b

            @pl.when(i >= 2)
            def _():
                wait_scatter(b)

            wait_idx(b)
            issue_gathers(b)

            @pl.when(i >= 1)
            def _():
                wait_gathers(1 - b)
                multiply(1 - b)
                issue_scatter(1 - b)

            @pl.when(i + 1 < NCH)
            def _():
                issue_idx(i + 1, 1 - b)

    wait_gathers(1)
    multiply(1)
    issue_scatter(1)
    wait_scatter(0)
    wait_scatter(1)

    if first:
        w2 = sid * NC + (1 - cid)

        def issue_didx(i, b):
            base = w2 * EPW + i * C
            pltpu.async_copy(dst_h.at[pl.ds(base, C)], dst_v.at[b], isem[b])

        def wait_didx(b):
            pltpu.make_async_copy(dst_h.at[pl.ds(0, C)], dst_v.at[b],
                                  isem[b]).wait()

        def issue_dscat(b):
            pltpu.async_copy(ones_v, deg_sh.at[dst_v.at[b]], dss[b],
                             add=True)

        def wait_dscat(b):
            pltpu.make_async_copy(inv_h.at[pl.ds(0, C)], ones_v,
                                  dss[b]).wait()

        issue_didx(0, 0)

        @pl.loop(0, NCH, step=2)
        def _(ii):
            for b in range(2):
                i = ii + b
                wait_didx(b)

                @pl.when(i >= 2)
                def _():
                    wait_dscat(b)

                issue_dscat(b)

                @pl.when(i + 1 < NCH)
                def _():
                    issue_didx(i + 1, 1 - b)

        wait_dscat(0)
        wait_dscat(1)

    plsc.subcore_barrier()

    @pl.loop(0, ZCH)
    def _(k):
        r0 = sid * RPS + k * C
        pltpu.sync_copy(agg_sh.at[pl.ds(r0, C)], xr_v.at[0])
        if first:
            pltpu.sync_copy(deg_sh.at[pl.ds(r0, C)], inv_v)

            @pl.loop(0, C, step=16)
            def _(j):
                inv_v[pl.ds(j, 16)] = 1.0 / jnp.maximum(
                    inv_v[pl.ds(j, 16)], 1.0)
        else:
            pltpu.sync_copy(inv_h.at[pl.ds(r0, C)], inv_v)

        @pl.loop(0, C, step=16)
        def _(g):
            iv16 = inv_v[pl.ds(g, 16)]

            @pl.loop(0, 16)
            def _(t):
                ivv = lax.gather(
                    iv16, jnp.full((16, 1), t, jnp.int32),
                    lax.GatherDimensionNumbers(
                        offset_dims=(), collapsed_slice_dims=(0,),
                        start_index_map=(0,)),
                    slice_sizes=(1,),
                    mode=lax.GatherScatterMode.PROMISE_IN_BOUNDS)

                for j in range(0, D, 16):
                    xr_v[0, g + t, pl.ds(j, 16)] = (
                        xr_v[0, g + t, pl.ds(j, 16)] * ivv)

        pltpu.sync_copy(xr_v.at[0], agg_out.at[cid, pl.ds(r0, C)])
        if first:
            @pl.when(cid == 0)
            def _():
                pltpu.sync_copy(inv_v, inv_out.at[pl.ds(r0, C)])


def _make_sc_layer(first):
    mesh = plsc.VectorSubcoreMesh(core_axis_name="c", subcore_axis_name="s")
    return pl.kernel(
        functools.partial(_sc_layer_body, first),
        out_type=[jax.ShapeDtypeStruct((NC, NPAD, D), jnp.float32),
                  jax.ShapeDtypeStruct((NPAD,), jnp.float32)],
        mesh=mesh,
        scratch_types=[
            pltpu.VMEM_SHARED((NPAD, D), jnp.float32),
            pltpu.VMEM_SHARED((NPAD,), jnp.float32),
            pltpu.VMEM((2, C), jnp.int32),
            pltpu.VMEM((2, C), jnp.int32),
            pltpu.VMEM((2, C), jnp.int32),
            pltpu.VMEM((2, C, D), jnp.float32),
            pltpu.VMEM((2, C, D), jnp.float32),
            pltpu.VMEM((C,), jnp.float32),
            pltpu.VMEM((C,), jnp.float32),
        ] + [pltpu.SemaphoreType.DMA] * 10,
    )


_sc_layer_first = _make_sc_layer(True)
_sc_layer_second = _make_sc_layer(False)


def _rel_body(rel_ref, wr1_ref, wr2_ref, rel2_ref, relout_ref):
    r2 = jnp.dot(rel_ref[...], wr1_ref[...], preferred_element_type=jnp.float32)
    rel2_ref[...] = r2
    relout_ref[...] = jnp.dot(r2, wr2_ref[...],
                              preferred_element_type=jnp.float32)


_rel_update = pl.pallas_call(
    _rel_body,
    out_shape=[jax.ShapeDtypeStruct((R, D), jnp.float32),
               jax.ShapeDtypeStruct((R, D), jnp.float32)],
)

BN = 1024
NB = NPAD // BN


def _dense_body(p_ref, xin_ref, w_ref, ws_ref, h_ref):
    agg = p_ref[0] + p_ref[1]
    xin = xin_ref[...]
    h = (jnp.dot(agg, w_ref[...], preferred_element_type=jnp.float32)
         + jnp.dot(xin, ws_ref[...], preferred_element_type=jnp.float32))
    h_ref[...] = jnp.maximum(h, 0.0) + xin


_dense_layer = pl.pallas_call(
    _dense_body,
    grid=(NB,),
    in_specs=[
        pl.BlockSpec((NC, BN, D), lambda i: (0, i, 0)),
        pl.BlockSpec((BN, D), lambda i: (i, 0)),
        pl.BlockSpec((D, D), lambda i: (0, 0)),
        pl.BlockSpec((D, D), lambda i: (0, 0)),
    ],
    out_specs=pl.BlockSpec((BN, D), lambda i: (i, 0)),
    out_shape=jax.ShapeDtypeStruct((NPAD, D), jnp.float32),
)


def kernel(x, edge_index, edge_type, relation_embs, W1, Ws1, Wr1, W2, Ws2, Wr2):
    pad = EPAD - E
    src = jnp.concatenate([edge_index[0], jnp.zeros((pad,), jnp.int32)])
    dst = jnp.concatenate([edge_index[1], jnp.full((pad,), N, jnp.int32)])
    typ = jnp.concatenate([edge_type, jnp.zeros((pad,), jnp.int32)])
    x_p = jnp.concatenate([x, jnp.zeros((NPAD - N, D), jnp.float32)])

    rel2, rel_out = _rel_update(relation_embs, Wr1, Wr2)

    dummy_inv = jnp.zeros((NPAD,), jnp.float32)
    p1, inv = _sc_layer_first(x_p, relation_embs, src, dst, typ, dummy_inv)
    h1 = _dense_layer(p1, x_p, W1, Ws1)

    p2, _ = _sc_layer_second(h1, rel2, src, dst, typ, inv)
    h2 = _dense_layer(p2, h1, W2, Ws2)

    return (h2[:N], rel_out)
